# baseline (speedup 1.0000x reference)
import functools

import jax
import jax.numpy as jnp
from jax import lax
from jax.experimental import pallas as pl
from jax.experimental.pallas import tpu as pltpu
from jax.experimental.pallas import tpu_sc as plsc

_C = 8
_PAIRS = [(i, j) for i in range(_C) for j in range(i + 1, _C)]
_DIM = len(_PAIRS)
_NC, _NS, _L = 2, 16, 16
_NW = _NC * _NS
_NPAR = _C + 3 + _DIM
_GW = _L * _C


@functools.lru_cache(maxsize=None)
def _build_sc_kernel(B: int):
    n_groups_total = B // _L
    groups_per_w = n_groups_total // _NW
    mesh = plsc.VectorSubcoreMesh(
        core_axis_name="c", subcore_axis_name="s",
        num_cores=_NC, num_subcores=_NS)

    rows_per_w = B // _NW

    @functools.partial(
        pl.kernel,
        mesh=mesh,
        compiler_params=pltpu.CompilerParams(needs_layout_passes=False),
        out_type=jax.ShapeDtypeStruct((_C, B), jnp.float32),
        scratch_types=[
            pltpu.VMEM((_C, rows_per_w), jnp.float32),
            pltpu.VMEM((_C, rows_per_w), jnp.float32),
            pltpu.VMEM((_NPAR, _L), jnp.float32),
        ],
    )
    def k(x_hbm, par_hbm, out_hbm, xv, ov, pv):
        wid = lax.axis_index("s") * _NC + lax.axis_index("c")
        b0 = wid * rows_per_w
        pltpu.sync_copy(x_hbm.at[:, pl.ds(b0, rows_per_w)], xv)
        pltpu.sync_copy(par_hbm, pv)

        d1 = [pv[c, :] for c in range(_C)]
        sign_bit = jnp.full((_L,), jnp.int32(-2147483648))
        rne_bias = jnp.full((_L,), jnp.int32(32767))
        bf_mask = jnp.full((_L,), jnp.int32(-65536))
        one = jnp.full((_L,), jnp.int32(1))

        def bf_round(x):
            u = plsc.bitcast(x, jnp.int32)
            r = (u + ((u >> 16) & one) + rne_bias) & bf_mask
            return plsc.bitcast(r, jnp.float32)

        w0 = bf_round(pv[_C, :])
        w1 = bf_round(pv[_C + 1, :])
        w2v = bf_round(pv[_C + 2, :])

        def body(g, carry):
            ch = []
            for c in range(_C):
                ch.append(bf_round(xv[c, pl.ds(g * _L, _L)] * d1[c]))
            acc = [None] * _C
            for p, (i, j) in enumerate(_PAIRS):
                d = ch[i] - ch[j]
                a = jnp.abs(d)
                sb = plsc.bitcast(d, jnp.int32) & sign_bit
                a2 = d * d
                v = (bf_round(a) * w0 + bf_round(a2) * w1
                     + bf_round(a2 * a) * w2v)
                v = jnp.maximum(v, v * jnp.float32(0.01))
                v = v * pv[_C + 3 + p, :]
                t = plsc.bitcast(plsc.bitcast(v, jnp.int32) ^ sb, jnp.float32)
                acc[i] = (-t) if acc[i] is None else (acc[i] - t)
                acc[j] = t if acc[j] is None else (acc[j] + t)
            for c in range(_C):
                ov[c, pl.ds(g * _L, _L)] = acc[c]
            return carry

        lax.fori_loop(0, groups_per_w, body, 0, unroll=False)
        pltpu.sync_copy(ov, out_hbm.at[:, pl.ds(b0, rows_per_w)])

    return k


def kernel(x, diag1, w2, diag3, diff_indices, i_idx, j_idx):
    B = x.shape[0]
    par = jnp.concatenate([diag1.astype(jnp.float32),
                           w2[0].astype(jnp.float32),
                           diag3.astype(jnp.float32)])
    par = jnp.broadcast_to(par[:, None], (_NPAR, _L))
    out = _build_sc_kernel(B)(x.T, par)
    return out.T[:, :, None]
